# baseline (speedup 1.0000x reference)
import functools

import jax
import jax.numpy as jnp
from jax import lax
from jax.experimental import pallas as pl
from jax.experimental.pallas import tpu as pltpu
from jax.experimental.pallas import tpu_sc as plsc

N_OBJECTS = 100000
NUM_VERTICES = 12
INPUT_DIM = 64
OUTPUT_DIM = 64
BATCH = 16384
ROW = NUM_VERTICES * INPUT_DIM

NC = 2
NS = 16
NW = NC * NS
PW_PER = ROW // NW
PB_PER = OUTPUT_DIM // NW
NPLANES = PW_PER + PB_PER
GSEG = 4096
GROUPS = GSEG // 16
UNROLL = 8
HSPLIT = 50048
HI_SZ = N_OBJECTS - HSPLIT

_mesh = plsc.VectorSubcoreMesh(core_axis_name="c", subcore_axis_name="s")


def _pass(idx_v, half, buf, q, lo):
    zeros = jnp.zeros((16,), jnp.int32)
    iota = lax.iota(jnp.int32, 16)

    @plsc.parallel_loop(0, GROUPS, unroll=UNROLL)
    def grp(g):
        off = g * 16
        ivec = idx_v[pl.ds(q * GSEG + off, 16)]
        if lo:
            m = ivec < HSPLIT
            isel = jnp.where(m, ivec, 0)
        else:
            m = ivec >= HSPLIT
            isel = jnp.where(m, ivec - HSPLIT, 0)
        vals = plsc.load_gather(half, [zeros, isel], mask=m)
        plsc.store_scatter(buf, [zeros, iota + off], vals, mask=m)


@functools.partial(
    pl.kernel,
    out_type=(
        jax.ShapeDtypeStruct((ROW, BATCH), jnp.float32),
        jax.ShapeDtypeStruct((OUTPUT_DIM, BATCH), jnp.float32),
    ),
    mesh=_mesh,
    compiler_params=pltpu.CompilerParams(needs_layout_passes=False),
    scratch_types=[
        pltpu.VMEM((BATCH,), jnp.int32),
        pltpu.VMEM((1, HSPLIT), jnp.float32),
        pltpu.VMEM((1, HI_SZ), jnp.float32),
        pltpu.VMEM((1, GSEG), jnp.float32),
        pltpu.VMEM((1, GSEG), jnp.float32),
        pltpu.SemaphoreType.DMA,
        pltpu.SemaphoreType.DMA,
        pltpu.SemaphoreType.DMA,
        pltpu.SemaphoreType.DMA,
    ],
)
def _gather_sc(w_hbm, b_hbm, idx_hbm, w_out, b_out,
               idx_v, lobuf, hibuf, outb0, outb1, sl, sh, os0, os1):
    wid = lax.axis_index("s") * NC + lax.axis_index("c")
    pltpu.sync_copy(idx_hbm, idx_v)
    bufs = (outb0, outb1)

    def stage(j, part, sem):
        lo_sl = pl.ds(0, HSPLIT)
        hi_sl = pl.ds(HSPLIT, HI_SZ)
        sl_ = lo_sl if part == 0 else hi_sl
        dbuf = lobuf if part == 0 else hibuf

        @pl.when(j < PW_PER)
        def _():
            p = wid * PW_PER + j
            pltpu.async_copy(w_hbm.at[pl.ds(p, 1)].at[:, sl_], dbuf, sem)

        @pl.when(jnp.logical_and(j >= PW_PER, j < NPLANES))
        def _():
            p = wid * PB_PER + (j - PW_PER)
            pltpu.async_copy(b_hbm.at[pl.ds(p, 1)].at[:, sl_], dbuf, sem)

    osems = (os0, os1)

    def out_dma(j, q, buf, sem):
        @pl.when(j < PW_PER)
        def _():
            p = wid * PW_PER + j
            pltpu.async_copy(buf, w_out.at[pl.ds(p, 1)].at[:, pl.ds(q * GSEG, GSEG)], sem)

        @pl.when(j >= PW_PER)
        def _():
            p = wid * PB_PER + (j - PW_PER)
            pltpu.async_copy(buf, b_out.at[pl.ds(p, 1)].at[:, pl.ds(q * GSEG, GSEG)], sem)

    def wait_out(sem):
        pltpu.make_async_copy(
            outb0, w_out.at[pl.ds(0, 1)].at[:, pl.ds(0, GSEG)], sem).wait()

    stage(0, 0, sl)
    stage(0, 1, sh)

    def plane(j, carry):
        pltpu.make_async_copy(
            w_hbm.at[pl.ds(0, 1)].at[:, pl.ds(0, HSPLIT)], lobuf, sl).wait()
        for half in range(2):
            qa = 2 * half
            for t in range(2):
                q = qa + t

                @pl.when(jnp.logical_or(j > 0, q >= 2))
                def _():
                    wait_out(osems[t])

                _pass(idx_v, lobuf, bufs[t], q, lo=True)
            if half == 0:
                pltpu.make_async_copy(
                    w_hbm.at[pl.ds(0, 1)].at[:, pl.ds(HSPLIT, HI_SZ)],
                    hibuf, sh).wait()
            for t in range(2):
                q = qa + t
                _pass(idx_v, hibuf, bufs[t], q, lo=False)
                out_dma(j, q, bufs[t], osems[t])
            if half == 1:
                stage(j + 1, 0, sl)
        stage(j + 1, 1, sh)
        return carry

    lax.fori_loop(0, NPLANES, plane, 0)
    for t in range(2):
        wait_out(osems[t])


def kernel(weight, bias, obj_ids):
    w2 = weight.transpose(1, 2, 0).reshape(ROW, N_OBJECTS)
    b2 = bias.transpose(1, 0)
    w_t, b_t = _gather_sc(w2, b2, obj_ids.astype(jnp.int32))
    w_sel = w_t.reshape(NUM_VERTICES, INPUT_DIM, BATCH).transpose(2, 0, 1)
    b_sel = b_t.transpose(1, 0)
    return w_sel, b_sel
